# baseline (speedup 1.0000x reference)
import functools

import jax
import jax.numpy as jnp
from jax import lax
from jax.experimental import pallas as pl
from jax.experimental.pallas import tpu as pltpu
from jax.experimental.pallas import tpu_sc as plsc

NC = 2
NS = 16
NW = NC * NS
LANES = 16

C = 128
NBUF = 2


@functools.partial(jax.jit, static_argnames=("B", "L", "D"))
def _embed(idx_flat, token_table, pos_flat, B, L, D):
    N = B * L
    per_w = N // NW
    nchunks = per_w // C
    scale = float(D) ** 0.5

    mesh = plsc.VectorSubcoreMesh(core_axis_name="c", subcore_axis_name="s")

    @functools.partial(
        pl.kernel,
        mesh=mesh,
        out_type=jax.ShapeDtypeStruct((N, D), jnp.float32),
        compiler_params=pltpu.CompilerParams(use_tc_tiling_on_sc=False),
        scratch_types=[
            pltpu.VMEM((NBUF, C), jnp.int32),
            pltpu.VMEM((NBUF, C, D), jnp.float32),
            pltpu.VMEM((NBUF, C, D), jnp.float32),
            pltpu.VMEM((2 * L * D,), jnp.float32),
            pltpu.SemaphoreType.DMA,
            pltpu.SemaphoreType.DMA,
            pltpu.SemaphoreType.DMA,
            pltpu.SemaphoreType.DMA,
        ],
    )
    def k(tab_hbm, idx_hbm, pos_hbm, out_hbm,
          idx_v, rows_g, rows_o, pos2_v, gs0, gs1, os0, os1):
        gsems = [gs0, gs1]
        osems = [os0, os1]
        wid = lax.axis_index("s") * NC + lax.axis_index("c")
        base = wid * per_w

        pltpu.sync_copy(pos_hbm, pos2_v.at[pl.ds(0, L * D)])
        pltpu.sync_copy(pos_hbm, pos2_v.at[pl.ds(L * D, L * D)])

        def start_gather(b, gg):
            row0 = base + gg * C
            pltpu.sync_copy(idx_hbm.at[pl.ds(row0, C)], idx_v.at[b])
            pltpu.make_async_copy(
                tab_hbm.at[idx_v.at[b]], rows_g.at[b], gsems[b]).start()

        for b in range(NBUF):
            start_gather(b, b)

        def chunk_body(g2, _):
            for b in range(NBUF):
                gg = g2 * NBUF + b
                row0 = base + gg * C
                pltpu.make_async_copy(
                    tab_hbm.at[idx_v.at[b]], rows_g.at[b], gsems[b]).wait()

                @pl.when(gg >= NBUF)
                def _():
                    pltpu.make_async_copy(
                        rows_o.at[b], out_hbm.at[pl.ds(row0, C)],
                        osems[b]).wait()

                phase = (row0 % L) * D

                def row_body(r, _):
                    for q in range(D // LANES):
                        col = q * LANES
                        v = rows_g[b, r, pl.ds(col, LANES)]
                        p = pos2_v[pl.ds(phase + r * D + col, LANES)]
                        rows_o[b, r, pl.ds(col, LANES)] = v * scale + p
                    return 0

                lax.fori_loop(0, C, row_body, 0)

                pltpu.make_async_copy(
                    rows_o.at[b], out_hbm.at[pl.ds(row0, C)],
                    osems[b]).start()

                @pl.when(gg + NBUF < nchunks)
                def _():
                    start_gather(b, gg + NBUF)
            return 0

        lax.fori_loop(0, nchunks // NBUF, chunk_body, 0)

        for b in range(NBUF):
            gg = nchunks - NBUF + b
            row0 = base + gg * C
            pltpu.make_async_copy(
                rows_o.at[b], out_hbm.at[pl.ds(row0, C)], osems[b]).wait()

    return k(token_table, idx_flat, pos_flat)


def kernel(input_word_ids, token_table, position_table):
    B, L = input_word_ids.shape
    V, D = token_table.shape
    idx_flat = input_word_ids.reshape(B * L).astype(jnp.int32)
    pos_flat = position_table.reshape(L * D)
    out = _embed(idx_flat, token_table, pos_flat, B, L, D)
    return out.reshape(B, L, D)
